# baseline (speedup 1.0000x reference)
import functools

import jax
import jax.numpy as jnp
from jax import lax
from jax.experimental import pallas as pl
from jax.experimental.pallas import tpu as pltpu
from jax.experimental.pallas import tpu_sc as plsc

_VOCAB = 100000
_HIDDEN = 128
_BATCH = 4096
_SEQ = 200
_PAD = 16
_NCLS = 2
_NW = 32
_BPW = _BATCH // _NW
_NBUF = 16
_PACKW = _HIDDEN // _PAD
_PROWS = _VOCAB // _PACKW
_ROW_BLK = 512
_GRID = -(-_PROWS // _ROW_BLK)
_PROWS_PAD = _GRID * _ROW_BLK


def _proj_body(table_ref, w_ref, b_ref, out_ref):
    out_ref[...] = (
        jax.lax.dot_general(
            table_ref[...], w_ref[...], (((1,), (0,)), ((), ())),
            preferred_element_type=jnp.float32,
        )
        + b_ref[...]
    )


def _project_table(table8, w_big, b_big):
    return pl.pallas_call(
        _proj_body,
        grid=(_GRID,),
        in_specs=[
            pl.BlockSpec((_ROW_BLK, _PACKW * _HIDDEN), lambda i: (i, 0)),
            pl.BlockSpec((_PACKW * _HIDDEN, _HIDDEN), lambda i: (0, 0)),
            pl.BlockSpec((1, _HIDDEN), lambda i: (0, 0)),
        ],
        out_specs=pl.BlockSpec((_ROW_BLK, _HIDDEN), lambda i: (i, 0)),
        out_shape=jax.ShapeDtypeStruct((_PROWS_PAD, _HIDDEN), jnp.float32),
    )(table8, w_big, b_big)


@functools.partial(
    pl.kernel,
    out_type=jax.ShapeDtypeStruct((_BATCH, _PAD), jnp.float32),
    mesh=plsc.VectorSubcoreMesh(core_axis_name="c", subcore_axis_name="s"),
    scratch_types=[
        pltpu.VMEM((_SEQ // 8, 8, _BPW), jnp.int32),
        pltpu.VMEM((_BPW, _PAD), jnp.float32),
        pltpu.SemaphoreType.DMA,
        pltpu.SemaphoreType.DMA,
    ],
    compiler_params=pltpu.CompilerParams(
        use_tc_tiling_on_sc=False, needs_layout_passes=False
    ),
)
def _sc_pool(ids_hbm, tp_hbm, out_hbm, ids_v, acc_v, sem0, sem):
    wid = lax.axis_index("s") * 2 + lax.axis_index("c")
    base = wid * _BPW

    pltpu.sync_copy(ids_hbm.at[:, wid], ids_v)

    pltpu.async_copy(tp_hbm.at[ids_v.at[0, 0]], acc_v, sem0).wait()

    @pl.loop(1, _SEQ)
    def _issue(l):
        @pl.when(l > _NBUF)
        def _():
            pltpu.make_async_copy(tp_hbm.at[ids_v.at[0, 0]], acc_v, sem).wait()

        pltpu.async_copy(
            tp_hbm.at[ids_v.at[l // 8, lax.rem(l, 8)]], acc_v, sem, add=True
        )

    @pl.loop(0, _NBUF)
    def _drain(_):
        pltpu.make_async_copy(tp_hbm.at[ids_v.at[0, 0]], acc_v, sem).wait()

    pltpu.sync_copy(acc_v, out_hbm.at[pl.ds(base, _BPW)])


def kernel(input_ids, embedding_table, fc_weight, fc_bias):
    ids = (
        input_ids.astype(jnp.int32)
        .T.reshape(_SEQ // 8, 8, _BATCH // _BPW, _BPW)
        .transpose(0, 2, 1, 3)
    )
    wpad = jnp.zeros((_HIDDEN, _PAD), jnp.float32)
    wpad = wpad.at[:, :_NCLS].set(fc_weight.T * jnp.float32(1.0 / _SEQ))
    w_big = jnp.kron(jnp.eye(_PACKW, dtype=jnp.float32), wpad)
    bpad = jnp.zeros((_PAD,), jnp.float32)
    bpad = bpad.at[:_NCLS].set(fc_bias * jnp.float32(1.0 / _SEQ))
    b_big = jnp.tile(bpad, _PACKW).reshape(1, _HIDDEN)

    table8 = embedding_table.reshape(_PROWS, _PACKW * _HIDDEN)
    tp = _project_table(table8, w_big, b_big).reshape(_PROWS_PAD * _PACKW, _PAD)
    out16 = _sc_pool(ids, tp)
    return out16[:, :_NCLS]


# speedup vs baseline: 1.6334x; 1.6334x over previous
import functools

import jax
import jax.numpy as jnp
from jax import lax
from jax.experimental import pallas as pl
from jax.experimental.pallas import tpu as pltpu
from jax.experimental.pallas import tpu_sc as plsc

_VOCAB = 100000
_HIDDEN = 128
_BATCH = 4096
_SEQ = 200
_PAD = 16
_NCLS = 2
_VPAD = 102400
_NW = 32
_BPW = _BATCH // _NW
_NBUF = 16
_ROW_BLK8 = 512


def _proj_body(table_ref, w_ref, b_ref, out_ref):
    w_scaled = w_ref[...] * jnp.float32(1.0 / _SEQ)
    bias = b_ref[...] * jnp.float32(1.0 / _SEQ)
    for j in range(8):
        psum = jax.lax.dot_general(
            table_ref[pl.ds(j * _ROW_BLK8, _ROW_BLK8), :], w_scaled,
            (((1,), (1,)), ((), ())),
            preferred_element_type=jnp.float32,
        )
        out_ref[:, j * _PAD : j * _PAD + _NCLS] = psum + bias


def _project_table(table, fc_weight, fc_bias):
    return pl.pallas_call(
        _proj_body,
        grid=(_VPAD // 4096,),
        in_specs=[
            pl.BlockSpec((8 * _ROW_BLK8, _HIDDEN), lambda i: (i, 0)),
            pl.BlockSpec((_NCLS, _HIDDEN), lambda i: (0, 0)),
            pl.BlockSpec((1, _NCLS), lambda i: (0, 0)),
        ],
        out_specs=pl.BlockSpec((_ROW_BLK8, 8 * _PAD), lambda i: (i, 0)),
        out_shape=jax.ShapeDtypeStruct((_VPAD // 8, 8 * _PAD), jnp.float32),
    )(table, fc_weight, fc_bias.reshape(1, _NCLS))


@functools.partial(
    pl.kernel,
    out_type=jax.ShapeDtypeStruct((_BATCH, _PAD), jnp.float32),
    mesh=plsc.VectorSubcoreMesh(core_axis_name="c", subcore_axis_name="s"),
    scratch_types=[
        pltpu.VMEM((_SEQ // 8, 8, _BPW), jnp.int32),
        pltpu.VMEM((_NBUF, _BPW), jnp.int32),
        pltpu.VMEM((_BPW, _PAD), jnp.float32),
        pltpu.SemaphoreType.DMA,
        pltpu.SemaphoreType.DMA,
    ],
    compiler_params=pltpu.CompilerParams(
        use_tc_tiling_on_sc=False, needs_layout_passes=False
    ),
)
def _sc_pool(ids_hbm, tp_hbm, out_hbm, ids_v, ring_v, acc_v, sem0, sem):
    wid = lax.axis_index("s") * 2 + lax.axis_index("c")
    base = wid * _BPW

    pltpu.sync_copy(ids_hbm.at[:, wid], ids_v)

    def build_col(l, slot):
        a = l // 8
        b = lax.rem(l, 8)
        dst = ring_v.at[slot]
        for g in range(_BPW // 16):
            v = ids_v[a, b, pl.ds(g * 16, 16)]
            u = (v & -4096) + ((v & 511) << 3) + ((v & 4095) >> 9)
            dst[pl.ds(g * 16, 16)] = u

    build_col(0, 0)
    pltpu.async_copy(tp_hbm.at[ring_v.at[0]], acc_v, sem0).wait()

    @pl.loop(1, _SEQ)
    def _issue(l):
        @pl.when(l > _NBUF)
        def _():
            pltpu.make_async_copy(tp_hbm.at[ring_v.at[0]], acc_v, sem).wait()

        slot = lax.rem(l, _NBUF)
        build_col(l, slot)
        pltpu.async_copy(tp_hbm.at[ring_v.at[slot]], acc_v, sem, add=True)

    @pl.loop(0, _NBUF)
    def _drain(_):
        pltpu.make_async_copy(tp_hbm.at[ring_v.at[0]], acc_v, sem).wait()

    pltpu.sync_copy(acc_v, out_hbm.at[pl.ds(base, _BPW)])


def kernel(input_ids, embedding_table, fc_weight, fc_bias):
    ids = (
        input_ids.astype(jnp.int32)
        .T.reshape(_SEQ // 8, 8, _BATCH // _BPW, _BPW)
        .transpose(0, 2, 1, 3)
    )
    tp = _project_table(embedding_table, fc_weight, fc_bias).reshape(_VPAD, _PAD)
    out16 = _sc_pool(ids, tp)
    return out16[:, :_NCLS]


# speedup vs baseline: 1.7896x; 1.0956x over previous
import functools

import jax
import jax.numpy as jnp
from jax import lax
from jax.experimental import pallas as pl
from jax.experimental.pallas import tpu as pltpu
from jax.experimental.pallas import tpu_sc as plsc

_VOCAB = 100000
_HIDDEN = 128
_BATCH = 4096
_SEQ = 200
_PAD = 16
_NCLS = 2
_VPAD = 106496
_NW = 32
_BPW = _BATCH // _NW
_NBUF = 16
_ROW_BLK8 = 1024


def _proj_body(table_ref, w_ref, b_ref, out_ref):
    w_scaled = w_ref[...] * jnp.float32(1.0 / _SEQ)
    bias = b_ref[...] * jnp.float32(1.0 / _SEQ)
    for j in range(8):
        psum = jax.lax.dot_general(
            table_ref[pl.ds(j * _ROW_BLK8, _ROW_BLK8), :], w_scaled,
            (((1,), (1,)), ((), ())),
            preferred_element_type=jnp.float32,
        )
        out_ref[:, j * _PAD : j * _PAD + _NCLS] = psum + bias


def _project_table(table, fc_weight, fc_bias):
    return pl.pallas_call(
        _proj_body,
        grid=(_VPAD // (8 * _ROW_BLK8),),
        in_specs=[
            pl.BlockSpec((8 * _ROW_BLK8, _HIDDEN), lambda i: (i, 0)),
            pl.BlockSpec((_NCLS, _HIDDEN), lambda i: (0, 0)),
            pl.BlockSpec((1, _NCLS), lambda i: (0, 0)),
        ],
        out_specs=pl.BlockSpec((_ROW_BLK8, 8 * _PAD), lambda i: (i, 0)),
        out_shape=jax.ShapeDtypeStruct((_VPAD // 8, 8 * _PAD), jnp.float32),
    )(table, fc_weight, fc_bias.reshape(1, _NCLS))


@functools.partial(
    pl.kernel,
    out_type=jax.ShapeDtypeStruct((_BATCH, _PAD), jnp.float32),
    mesh=plsc.VectorSubcoreMesh(core_axis_name="c", subcore_axis_name="s"),
    scratch_types=[
        pltpu.VMEM((_SEQ // 8, 8, _BPW), jnp.int32),
        pltpu.VMEM((_NBUF, _BPW), jnp.int32),
        pltpu.VMEM((_BPW, _PAD), jnp.float32),
        pltpu.SemaphoreType.DMA,
        pltpu.SemaphoreType.DMA,
    ],
    compiler_params=pltpu.CompilerParams(
        use_tc_tiling_on_sc=False, needs_layout_passes=False
    ),
)
def _sc_pool(ids_hbm, tp_hbm, out_hbm, ids_v, ring_v, acc_v, sem0, sem):
    wid = lax.axis_index("s") * 2 + lax.axis_index("c")
    base = wid * _BPW

    pltpu.sync_copy(ids_hbm.at[:, wid], ids_v)

    def build_col(l, slot):
        a = l // 8
        b = lax.rem(l, 8)
        dst = ring_v.at[slot]
        for g in range(_BPW // 16):
            v = ids_v[a, b, pl.ds(g * 16, 16)]
            u = (v & -8192) + ((v & 1023) << 3) + ((v & 8191) >> 10)
            dst[pl.ds(g * 16, 16)] = u

    build_col(0, 0)
    pltpu.async_copy(tp_hbm.at[ring_v.at[0]], acc_v, sem0).wait()

    @pl.loop(1, _SEQ)
    def _issue(l):
        @pl.when(l > _NBUF)
        def _():
            pltpu.make_async_copy(tp_hbm.at[ring_v.at[0]], acc_v, sem).wait()

        slot = lax.rem(l, _NBUF)
        build_col(l, slot)
        pltpu.async_copy(tp_hbm.at[ring_v.at[slot]], acc_v, sem, add=True)

    @pl.loop(0, _NBUF)
    def _drain(_):
        pltpu.make_async_copy(tp_hbm.at[ring_v.at[0]], acc_v, sem).wait()

    pltpu.sync_copy(acc_v, out_hbm.at[pl.ds(base, _BPW)])


def kernel(input_ids, embedding_table, fc_weight, fc_bias):
    ids = (
        input_ids.astype(jnp.int32)
        .T.reshape(_SEQ // 8, 8, _BATCH // _BPW, _BPW)
        .transpose(0, 2, 1, 3)
    )
    tp = _project_table(embedding_table, fc_weight, fc_bias).reshape(_VPAD, _PAD)
    out16 = _sc_pool(ids, tp)
    return out16[:, :_NCLS]


# speedup vs baseline: 1.8158x; 1.0147x over previous
import functools

import jax
import jax.numpy as jnp
from jax import lax
from jax.experimental import pallas as pl
from jax.experimental.pallas import tpu as pltpu
from jax.experimental.pallas import tpu_sc as plsc

_VOCAB = 100000
_HIDDEN = 128
_BATCH = 4096
_SEQ = 200
_PAD = 16
_NCLS = 2
_VPAD = 114688
_NW = 32
_BPW = _BATCH // _NW
_NBUF = 16
_ROW_BLK8 = 2048


def _proj_body(table_ref, w_ref, b_ref, out_ref):
    w_scaled = w_ref[...] * jnp.float32(1.0 / _SEQ)
    bias = b_ref[...] * jnp.float32(1.0 / _SEQ)
    for j in range(8):
        psum = jax.lax.dot_general(
            table_ref[pl.ds(j * _ROW_BLK8, _ROW_BLK8), :], w_scaled,
            (((1,), (1,)), ((), ())),
            preferred_element_type=jnp.float32,
        )
        out_ref[:, j * _PAD : j * _PAD + _NCLS] = psum + bias


def _project_table(table, fc_weight, fc_bias):
    return pl.pallas_call(
        _proj_body,
        grid=(_VPAD // (8 * _ROW_BLK8),),
        in_specs=[
            pl.BlockSpec((8 * _ROW_BLK8, _HIDDEN), lambda i: (i, 0)),
            pl.BlockSpec((_NCLS, _HIDDEN), lambda i: (0, 0)),
            pl.BlockSpec((1, _NCLS), lambda i: (0, 0)),
        ],
        out_specs=pl.BlockSpec((_ROW_BLK8, 8 * _PAD), lambda i: (i, 0)),
        out_shape=jax.ShapeDtypeStruct((_VPAD // 8, 8 * _PAD), jnp.float32),
    )(table, fc_weight, fc_bias.reshape(1, _NCLS))


@functools.partial(
    pl.kernel,
    out_type=jax.ShapeDtypeStruct((_BATCH, _PAD), jnp.float32),
    mesh=plsc.VectorSubcoreMesh(core_axis_name="c", subcore_axis_name="s"),
    scratch_types=[
        pltpu.VMEM((_SEQ // 8, 8, _BPW), jnp.int32),
        pltpu.VMEM((_NBUF, _BPW), jnp.int32),
        pltpu.VMEM((_BPW, _PAD), jnp.float32),
        pltpu.SemaphoreType.DMA,
        pltpu.SemaphoreType.DMA,
    ],
    compiler_params=pltpu.CompilerParams(
        use_tc_tiling_on_sc=False, needs_layout_passes=False
    ),
)
def _sc_pool(ids_hbm, tp_hbm, out_hbm, ids_v, ring_v, acc_v, sem0, sem):
    wid = lax.axis_index("s") * 2 + lax.axis_index("c")
    base = wid * _BPW

    pltpu.sync_copy(ids_hbm.at[:, wid], ids_v)

    def build_col(l, slot):
        a = l // 8
        b = lax.rem(l, 8)
        dst = ring_v.at[slot]
        for g in range(_BPW // 16):
            v = ids_v[a, b, pl.ds(g * 16, 16)]
            u = (v & -16384) + ((v & 2047) << 3) + ((v & 16383) >> 11)
            dst[pl.ds(g * 16, 16)] = u

    build_col(0, 0)
    pltpu.async_copy(tp_hbm.at[ring_v.at[0]], acc_v, sem0).wait()

    @pl.loop(1, _SEQ)
    def _issue(l):
        @pl.when(l > _NBUF)
        def _():
            pltpu.make_async_copy(tp_hbm.at[ring_v.at[0]], acc_v, sem).wait()

        slot = lax.rem(l, _NBUF)
        build_col(l, slot)
        pltpu.async_copy(tp_hbm.at[ring_v.at[slot]], acc_v, sem, add=True)

    @pl.loop(0, _NBUF)
    def _drain(_):
        pltpu.make_async_copy(tp_hbm.at[ring_v.at[0]], acc_v, sem).wait()

    pltpu.sync_copy(acc_v, out_hbm.at[pl.ds(base, _BPW)])


def kernel(input_ids, embedding_table, fc_weight, fc_bias):
    ids = (
        input_ids.astype(jnp.int32)
        .T.reshape(_SEQ // 8, 8, _BATCH // _BPW, _BPW)
        .transpose(0, 2, 1, 3)
    )
    tp = _project_table(embedding_table, fc_weight, fc_bias).reshape(_VPAD, _PAD)
    out16 = _sc_pool(ids, tp)
    return out16[:, :_NCLS]
